# Initial kernel scaffold; baseline (speedup 1.0000x reference)
import jax
import jax.numpy as jnp
from jax.experimental import pallas as pl


def kernel(x, edge_index, edge_weight, edge_attr, aw1_w, aw1_b, d1_w, d1_b, d2_w, d2_b, l1_w, l2_w, l2_b, aw2_w, aw2_b):
    raise NotImplementedError("write your pallas kernel here")


# speedup vs baseline: 1.6012x; 1.6012x over previous
import functools
import math

import jax
import jax.numpy as jnp
from jax import lax
from jax.experimental import pallas as pl
from jax.experimental.pallas import tpu as pltpu
from jax.experimental.pallas import tpu_sc as plsc

CUTOFF = 10.0
LOG2 = math.log(2.0)

NW = 32
LANES = 16
CH = 2560


def _ssp(t):
    return jnp.maximum(t, 0.0) + jnp.log1p(jnp.exp(-jnp.abs(t))) - LOG2


def _ht_body(x_ref, aw1w_ref, aw1b_ref, l1w_ref, o_ref):
    t = lax.dot_general(x_ref[...], aw1w_ref[...], (((1,), (1,)), ((), ())),
                        preferred_element_type=jnp.float32) + aw1b_ref[...]
    o_ref[...] = lax.dot_general(l1w_ref[...], t, (((1,), (1,)), ((), ())),
                                 preferred_element_type=jnp.float32)


def _compute_ht(x, aw1_w, aw1_b, l1_w):
    n, h = x.shape
    return pl.pallas_call(
        _ht_body,
        out_shape=jax.ShapeDtypeStruct((h, n), jnp.float32),
    )(x, aw1_w, aw1_b.reshape(1, h), l1_w)


def _wt_body(ea_ref, ew_ref, d1w_ref, d1b_ref, d2w_ref, d2b_ref, o_ref):
    t = lax.dot_general(ea_ref[...], d1w_ref[...], (((1,), (1,)), ((), ())),
                        preferred_element_type=jnp.float32) + d1b_ref[...]
    t = _ssp(t)
    wt = lax.dot_general(d2w_ref[...], t, (((1,), (1,)), ((), ())),
                         preferred_element_type=jnp.float32) + d2b_ref[...]
    c = 0.5 * (jnp.cos(ew_ref[...] * (jnp.pi / CUTOFF)) + 1.0)
    o_ref[...] = wt * c


def _compute_wt(edge_attr, edge_weight, d1_w, d1_b, d2_w, d2_b, be):
    e, g = edge_attr.shape
    f = d1_w.shape[0]
    return pl.pallas_call(
        _wt_body,
        grid=(e // be,),
        in_specs=[
            pl.BlockSpec((be, g), lambda i: (i, 0)),
            pl.BlockSpec((1, be), lambda i: (0, i)),
            pl.BlockSpec((f, g), lambda i: (0, 0)),
            pl.BlockSpec((1, f), lambda i: (0, 0)),
            pl.BlockSpec((f, f), lambda i: (0, 0)),
            pl.BlockSpec((f, 1), lambda i: (0, 0)),
        ],
        out_specs=pl.BlockSpec((f, be), lambda i: (0, i)),
        out_shape=jax.ShapeDtypeStruct((f, e), jnp.float32),
    )(edge_attr, edge_weight.reshape(1, e), d1_w, d1_b.reshape(1, f),
      d2_w, d2_b.reshape(f, 1))


def _make_scatter(n, e, h):
    f_per = h // NW
    nchunks = e // CH
    mesh = plsc.VectorSubcoreMesh(core_axis_name="c", subcore_axis_name="s")

    @functools.partial(
        pl.kernel,
        out_type=jax.ShapeDtypeStruct((h * n,), jnp.float32),
        mesh=mesh,
        scratch_types=[
            pltpu.VMEM((f_per * n,), jnp.float32),
            pltpu.VMEM((f_per * n,), jnp.float32),
            pltpu.VMEM((CH,), jnp.int32),
            pltpu.VMEM((CH,), jnp.int32),
            pltpu.VMEM((2 * f_per, CH), jnp.float32),
        ],
        compiler_params=pltpu.CompilerParams(needs_layout_passes=False),
    )
    def scatter_kernel(ht_hbm, wt_hbm, src_hbm, dst_hbm, out_hbm,
                       h_v, a_v, src_v, dst_v, w_v):
        wid = lax.axis_index("s") * 2 + lax.axis_index("c")
        parity = wid % 2
        rowbase = parity * f_per
        slab0 = pl.multiple_of((wid - parity) * f_per, 2 * f_per)
        hoff = pl.multiple_of(wid * (f_per * n), 8)
        pltpu.sync_copy(ht_hbm.at[pl.ds(hoff, f_per * n)], h_v)

        def zero_body(i, carry):
            b = pl.multiple_of(i * LANES, LANES)
            a_v[pl.ds(b, LANES)] = jnp.zeros((LANES,), jnp.float32)
            return carry

        lax.fori_loop(0, (f_per * n) // LANES, zero_body, 0)

        def chunk_body(ci, carry):
            e0 = pl.multiple_of(ci * CH, CH)
            pltpu.sync_copy(src_hbm.at[pl.ds(e0, CH)], src_v)
            pltpu.sync_copy(dst_hbm.at[pl.ds(e0, CH)], dst_v)
            pltpu.sync_copy(wt_hbm.at[pl.ds(slab0, 2 * f_per), pl.ds(e0, CH)],
                            w_v)

            def group_body(g, gcarry):
                b = pl.multiple_of(g * LANES, LANES)
                sidx = src_v[pl.ds(b, LANES)]
                didx = dst_v[pl.ds(b, LANES)]
                for f in range(f_per):
                    hrow = plsc.load_gather(h_v, [sidx + (f * n)])
                    wrow = w_v[rowbase + f, pl.ds(b, LANES)]
                    plsc.addupdate_scatter(a_v, [didx + (f * n)], wrow * hrow)
                return gcarry

            lax.fori_loop(0, CH // LANES, group_body, 0)
            return carry

        lax.fori_loop(0, nchunks, chunk_body, 0)
        pltpu.sync_copy(a_v, out_hbm.at[pl.ds(hoff, f_per * n)])

    return scatter_kernel


def _out_body(x_ref, at_ref, l2w_ref, l2b_ref, aw2w_ref, aw2b_ref, o_ref):
    conv = lax.dot_general(at_ref[...], l2w_ref[...], (((0,), (1,)), ((), ())),
                           preferred_element_type=jnp.float32) + l2b_ref[...]
    s = _ssp(conv)
    o_ref[...] = (lax.dot_general(s, aw2w_ref[...], (((1,), (1,)), ((), ())),
                                  preferred_element_type=jnp.float32)
                  + aw2b_ref[...] + x_ref[...])


def _compute_out(x, aggr_t, l2_w, l2_b, aw2_w, aw2_b):
    n, h = x.shape
    return pl.pallas_call(
        _out_body,
        out_shape=jax.ShapeDtypeStruct((n, h), jnp.float32),
    )(x, aggr_t, l2_w, l2_b.reshape(1, h), aw2_w, aw2_b.reshape(1, h))


def kernel(x, edge_index, edge_weight, edge_attr,
           aw1_w, aw1_b, d1_w, d1_b, d2_w, d2_b,
           l1_w, l2_w, l2_b, aw2_w, aw2_b):
    n, h = x.shape
    e = edge_attr.shape[0]
    src = edge_index[0].astype(jnp.int32)
    dst = edge_index[1].astype(jnp.int32)

    ht = _compute_ht(x, aw1_w, aw1_b, l1_w)
    wt = _compute_wt(edge_attr, edge_weight, d1_w, d1_b, d2_w, d2_b, be=6400)
    aggr_flat = _make_scatter(n, e, h)(ht.reshape(h * n), wt, src, dst)
    return _compute_out(x, aggr_flat.reshape(h, n), l2_w, l2_b, aw2_w, aw2_b)


# speedup vs baseline: 3.4300x; 2.1421x over previous
import functools
import math

import jax
import jax.numpy as jnp
from jax import lax
from jax.experimental import pallas as pl
from jax.experimental.pallas import tpu as pltpu
from jax.experimental.pallas import tpu_sc as plsc

CUTOFF = 10.0
LOG2 = math.log(2.0)

NW = 32
LANES = 16
CH = 1280


def _ssp(t):
    return jnp.maximum(t, 0.0) + jnp.log1p(jnp.exp(-jnp.abs(t))) - LOG2


def _ht_body(x_ref, aw1w_ref, aw1b_ref, l1w_ref, o_ref):
    t = lax.dot_general(x_ref[...], aw1w_ref[...], (((1,), (1,)), ((), ())),
                        preferred_element_type=jnp.float32) + aw1b_ref[...]
    o_ref[...] = lax.dot_general(l1w_ref[...], t, (((1,), (1,)), ((), ())),
                                 preferred_element_type=jnp.float32)


def _compute_ht(x, aw1_w, aw1_b, l1_w):
    n, h = x.shape
    return pl.pallas_call(
        _ht_body,
        out_shape=jax.ShapeDtypeStruct((h, n), jnp.float32),
    )(x, aw1_w, aw1_b.reshape(1, h), l1_w)


def _wt_body(ea_ref, ew_ref, d1w_ref, d1b_ref, d2w_ref, d2b_ref, o_ref):
    t = lax.dot_general(ea_ref[...], d1w_ref[...], (((1,), (1,)), ((), ())),
                        preferred_element_type=jnp.float32) + d1b_ref[...]
    t = _ssp(t)
    wt = lax.dot_general(d2w_ref[...], t, (((1,), (1,)), ((), ())),
                         preferred_element_type=jnp.float32) + d2b_ref[...]
    c = 0.5 * (jnp.cos(ew_ref[...] * (jnp.pi / CUTOFF)) + 1.0)
    o_ref[...] = wt * c


def _compute_wt(edge_attr, edge_weight, d1_w, d1_b, d2_w, d2_b, be):
    e, g = edge_attr.shape
    f = d1_w.shape[0]
    return pl.pallas_call(
        _wt_body,
        grid=(e // be,),
        in_specs=[
            pl.BlockSpec((be, g), lambda i: (i, 0)),
            pl.BlockSpec((1, be), lambda i: (0, i)),
            pl.BlockSpec((f, g), lambda i: (0, 0)),
            pl.BlockSpec((1, f), lambda i: (0, 0)),
            pl.BlockSpec((f, f), lambda i: (0, 0)),
            pl.BlockSpec((f, 1), lambda i: (0, 0)),
        ],
        out_specs=pl.BlockSpec((f, be), lambda i: (0, i)),
        out_shape=jax.ShapeDtypeStruct((f, e), jnp.float32),
    )(edge_attr, edge_weight.reshape(1, e), d1_w, d1_b.reshape(1, f),
      d2_w, d2_b.reshape(f, 1))


def _make_scatter(n, e, h):
    f_per = h // NW
    nchunks = e // CH
    mesh = plsc.VectorSubcoreMesh(core_axis_name="c", subcore_axis_name="s")

    @functools.partial(
        pl.kernel,
        out_type=jax.ShapeDtypeStruct((h * n,), jnp.float32),
        mesh=mesh,
        scratch_types=[
            pltpu.VMEM((f_per * n,), jnp.float32),
            pltpu.VMEM((f_per * n,), jnp.float32),
            pltpu.VMEM((2, CH), jnp.int32),
            pltpu.VMEM((2, CH), jnp.int32),
            pltpu.VMEM((2, 2 * f_per, CH), jnp.float32),
            pltpu.SemaphoreType.DMA,
            pltpu.SemaphoreType.DMA,
        ],
        compiler_params=pltpu.CompilerParams(needs_layout_passes=False),
    )
    def scatter_kernel(ht_hbm, wt_hbm, src_hbm, dst_hbm, out_hbm,
                       h_v, a_v, src_v, dst_v, w_v, sem0, sem1):
        wid = lax.axis_index("s") * 2 + lax.axis_index("c")
        parity = wid % 2
        rowbase = parity * f_per
        slab0 = pl.multiple_of((wid - parity) * f_per, 2 * f_per)
        hoff = pl.multiple_of(wid * (f_per * n), 8)
        sems = (sem0, sem1)

        def issue(ci, b):
            e0 = pl.multiple_of(ci * CH, CH)
            pltpu.async_copy(src_hbm.at[pl.ds(e0, CH)], src_v.at[b], sems[b])
            pltpu.async_copy(dst_hbm.at[pl.ds(e0, CH)], dst_v.at[b], sems[b])
            pltpu.async_copy(
                wt_hbm.at[pl.ds(slab0, 2 * f_per), pl.ds(e0, CH)],
                w_v.at[b], sems[b])

        def drain(b):
            pltpu.make_async_copy(src_hbm.at[pl.ds(0, CH)], src_v.at[b],
                                  sems[b]).wait()
            pltpu.make_async_copy(dst_hbm.at[pl.ds(0, CH)], dst_v.at[b],
                                  sems[b]).wait()
            pltpu.make_async_copy(
                wt_hbm.at[pl.ds(0, 2 * f_per), pl.ds(0, CH)],
                w_v.at[b], sems[b]).wait()

        def compute(b):
            @plsc.parallel_loop(0, CH, LANES, unroll=4)
            def _(off):
                sidx = src_v[b, pl.ds(off, LANES)]
                didx = dst_v[b, pl.ds(off, LANES)]
                for f in range(f_per):
                    hrow = plsc.load_gather(h_v, [sidx + (f * n)])
                    wrow = w_v[b, rowbase + f, pl.ds(off, LANES)]
                    plsc.addupdate_scatter(a_v, [didx + (f * n)], wrow * hrow)

        pltpu.sync_copy(ht_hbm.at[pl.ds(hoff, f_per * n)], h_v)

        @plsc.parallel_loop(0, f_per * n, LANES, unroll=8)
        def _(off):
            a_v[pl.ds(off, LANES)] = jnp.zeros((LANES,), jnp.float32)

        issue(0, 0)

        def pair_body(p, carry):
            ci = p * 2
            issue(ci + 1, 1)
            drain(0)
            compute(0)

            @pl.when(ci + 2 < nchunks)
            def _():
                issue(ci + 2, 0)

            drain(1)
            compute(1)
            return carry

        lax.fori_loop(0, nchunks // 2, pair_body, 0)
        pltpu.sync_copy(a_v, out_hbm.at[pl.ds(hoff, f_per * n)])

    return scatter_kernel


def _out_body(x_ref, at_ref, l2w_ref, l2b_ref, aw2w_ref, aw2b_ref, o_ref):
    conv = lax.dot_general(at_ref[...], l2w_ref[...], (((0,), (1,)), ((), ())),
                           preferred_element_type=jnp.float32) + l2b_ref[...]
    s = _ssp(conv)
    o_ref[...] = (lax.dot_general(s, aw2w_ref[...], (((1,), (1,)), ((), ())),
                                  preferred_element_type=jnp.float32)
                  + aw2b_ref[...] + x_ref[...])


def _compute_out(x, aggr_t, l2_w, l2_b, aw2_w, aw2_b):
    n, h = x.shape
    return pl.pallas_call(
        _out_body,
        out_shape=jax.ShapeDtypeStruct((n, h), jnp.float32),
    )(x, aggr_t, l2_w, l2_b.reshape(1, h), aw2_w, aw2_b.reshape(1, h))


def kernel(x, edge_index, edge_weight, edge_attr,
           aw1_w, aw1_b, d1_w, d1_b, d2_w, d2_b,
           l1_w, l2_w, l2_b, aw2_w, aw2_b):
    n, h = x.shape
    e = edge_attr.shape[0]
    src = edge_index[0].astype(jnp.int32)
    dst = edge_index[1].astype(jnp.int32)

    ht = _compute_ht(x, aw1_w, aw1_b, l1_w)
    wt = _compute_wt(edge_attr, edge_weight, d1_w, d1_b, d2_w, d2_b, be=6400)
    aggr_flat = _make_scatter(n, e, h)(ht.reshape(h * n), wt, src, dst)
    return _compute_out(x, aggr_flat.reshape(h, n), l2_w, l2_b, aw2_w, aw2_b)
